# baseline (speedup 1.0000x reference)
import jax
import jax.numpy as jnp
from jax import lax
from jax.experimental import pallas as pl
from jax.experimental.pallas import tpu as pltpu
from jax.experimental.pallas import tpu_sc as plsc

_BATCH = 4
_SEQ = 4096
_HID = 4096
_LANES = 16
_NVREG = _SEQ // _LANES


def _sc_body(ids_hbm, hs_hbm, out_hbm, ids_v):
    b = lax.axis_index("s")
    pltpu.sync_copy(ids_hbm.at[b], ids_v)
    lane = lax.iota(jnp.int32, _LANES)

    def scan_body(j, carry):
        v = ids_v[pl.ds(j * _LANES, _LANES)]
        gi = lane + j * _LANES
        return jnp.minimum(carry, jnp.where(v == 0, gi, _SEQ))

    mvec = lax.fori_loop(
        0, _NVREG, scan_body, jnp.full((_LANES,), _SEQ, jnp.int32),
        unroll=8,
    )
    for sh in (1, 2, 4, 8):
        mvec = jnp.minimum(
            mvec, mvec.at[lane ^ sh].get(mode="promise_in_bounds")
        )
    p = mvec[0]
    idx = (p + (_SEQ - 1)) % _SEQ
    pltpu.sync_copy(hs_hbm.at[b, idx], out_hbm.at[b])


def kernel(input_ids, hidden_states):
    mesh = plsc.VectorSubcoreMesh(
        core_axis_name="c", subcore_axis_name="s",
        num_cores=1, num_subcores=_BATCH,
    )
    k = pl.kernel(
        _sc_body,
        out_type=jax.ShapeDtypeStruct((_BATCH, _HID), jnp.float32),
        mesh=mesh,
        scratch_types=[
            pltpu.VMEM((_SEQ,), jnp.int32),
        ],
    )
    return k(input_ids.astype(jnp.int32), hidden_states)


# speedup vs baseline: 1.0118x; 1.0118x over previous
import jax
import jax.numpy as jnp
from jax import lax
from jax.experimental import pallas as pl
from jax.experimental.pallas import tpu as pltpu
from jax.experimental.pallas import tpu_sc as plsc

_BATCH = 4
_SEQ = 4096
_HID = 4096
_LANES = 16
_NVREG = _SEQ // _LANES


def _sc_body(ids_hbm, hs_hbm, out_hbm, ids_v):
    b = lax.axis_index("s")

    @pl.when(b < _BATCH)
    def _():
        pltpu.sync_copy(ids_hbm.at[b], ids_v)
        lane = lax.iota(jnp.int32, _LANES)

        def scan_body(j, carry):
            v = ids_v[pl.ds(j * _LANES, _LANES)]
            gi = lane + j * _LANES
            return jnp.minimum(carry, jnp.where(v == 0, gi, _SEQ))

        mvec = lax.fori_loop(
            0, _NVREG, scan_body, jnp.full((_LANES,), _SEQ, jnp.int32),
            unroll=8,
        )
        for sh in (1, 2, 4, 8):
            mvec = jnp.minimum(
                mvec, mvec.at[lane ^ sh].get(mode="promise_in_bounds")
            )
        p = mvec[0]
        idx = (p + (_SEQ - 1)) % _SEQ
        pltpu.sync_copy(hs_hbm.at[b, idx], out_hbm.at[b])


def kernel(input_ids, hidden_states):
    mesh = plsc.VectorSubcoreMesh(
        core_axis_name="c", subcore_axis_name="s", num_cores=1,
    )
    k = pl.kernel(
        _sc_body,
        out_type=jax.ShapeDtypeStruct((_BATCH, _HID), jnp.float32),
        mesh=mesh,
        scratch_types=[
            pltpu.VMEM((_SEQ,), jnp.int32),
        ],
    )
    return k(input_ids.astype(jnp.int32), hidden_states)


# speedup vs baseline: 1.0759x; 1.0634x over previous
import jax
import jax.numpy as jnp
from jax import lax
from jax.experimental import pallas as pl
from jax.experimental.pallas import tpu as pltpu
from jax.experimental.pallas import tpu_sc as plsc

_BATCH = 4
_SEQ = 4096
_HID = 4096
_LANES = 16
_NSEG = 4
_CHUNK = _SEQ // _NSEG
_NVREG = _CHUNK // _LANES


def _sc_body(ids_hbm, hs_hbm, out_hbm, ids_v, cand_v, gath_v, row_v, shared):
    t = lax.axis_index("s")
    b = t // _NSEG
    seg = t % _NSEG
    lane = lax.iota(jnp.int32, _LANES)

    pltpu.sync_copy(ids_hbm.at[b, pl.ds(seg * _CHUNK, _CHUNK)], ids_v)
    base = seg * _CHUNK

    def scan_body(j, carry):
        v = ids_v[pl.ds(j * _LANES, _LANES)]
        gi = lane + (base + j * _LANES)
        return jnp.minimum(carry, jnp.where(v == 0, gi, _SEQ))

    mvec = lax.fori_loop(
        0, _NVREG, scan_body, jnp.full((_LANES,), _SEQ, jnp.int32),
        unroll=8,
    )
    cand_v[...] = mvec
    pltpu.sync_copy(cand_v, shared.at[t])
    plsc.subcore_barrier()

    @pl.when(t < _BATCH)
    def _():
        pltpu.sync_copy(shared.at[pl.ds(t * _NSEG, _NSEG)], gath_v)
        m = jnp.minimum(
            jnp.minimum(gath_v[0], gath_v[1]),
            jnp.minimum(gath_v[2], gath_v[3]),
        )
        for sh in (1, 2, 4, 8):
            m = jnp.minimum(m, m.at[lane ^ sh].get(mode="promise_in_bounds"))
        p = m[0]
        idx = (p + (_SEQ - 1)) % _SEQ
        pltpu.sync_copy(hs_hbm.at[t, idx], row_v)
        pltpu.sync_copy(row_v, out_hbm.at[t])


def kernel(input_ids, hidden_states):
    mesh = plsc.VectorSubcoreMesh(
        core_axis_name="c", subcore_axis_name="s", num_cores=1,
    )
    k = pl.kernel(
        _sc_body,
        out_type=jax.ShapeDtypeStruct((_BATCH, _HID), jnp.float32),
        mesh=mesh,
        scratch_types=[
            pltpu.VMEM((_CHUNK,), jnp.int32),
            pltpu.VMEM((_LANES,), jnp.int32),
            pltpu.VMEM((_NSEG, _LANES), jnp.int32),
            pltpu.VMEM((_HID,), jnp.float32),
            pltpu.VMEM_SHARED((_LANES, _LANES), jnp.int32),
        ],
    )
    return k(input_ids.astype(jnp.int32), hidden_states)


# speedup vs baseline: 1.0952x; 1.0179x over previous
import jax
import jax.numpy as jnp
from jax import lax
from jax.experimental import pallas as pl
from jax.experimental.pallas import tpu as pltpu
from jax.experimental.pallas import tpu_sc as plsc

_BATCH = 4
_SEQ = 4096
_HID = 4096
_LANES = 16
_HSEQ = _SEQ // 2
_HHID = _HID // 2
_NVREG_H = _HSEQ // _LANES


def _sc_body(ids_hbm, hs_hbm, out_hbm, ids_v, row_v, s0, s1, s2, s3):
    b = lax.axis_index("s")

    @pl.when(b < _BATCH)
    def _():
        c0 = pltpu.make_async_copy(
            ids_hbm.at[b, pl.ds(0, _HSEQ)], ids_v.at[pl.ds(0, _HSEQ)], s0
        )
        c0.start()
        c1 = pltpu.make_async_copy(
            ids_hbm.at[b, pl.ds(_HSEQ, _HSEQ)],
            ids_v.at[pl.ds(_HSEQ, _HSEQ)],
            s1,
        )
        c1.start()
        lane = lax.iota(jnp.int32, _LANES)

        def scan_half(base):
            def scan_body(j, carry):
                v = ids_v[pl.ds(base + j * _LANES, _LANES)]
                gi = lane + (base + j * _LANES)
                return jnp.minimum(carry, jnp.where(v == 0, gi, _SEQ))

            return scan_body

        c0.wait()
        mvec = lax.fori_loop(
            0, _NVREG_H, scan_half(0),
            jnp.full((_LANES,), _SEQ, jnp.int32), unroll=8,
        )
        c1.wait()
        mvec = lax.fori_loop(
            0, _NVREG_H, scan_half(_HSEQ), mvec, unroll=8,
        )
        for sh in (1, 2, 4, 8):
            mvec = jnp.minimum(
                mvec, mvec.at[lane ^ sh].get(mode="promise_in_bounds")
            )
        p = mvec[0]
        idx = (p + (_SEQ - 1)) % _SEQ
        g0 = pltpu.make_async_copy(
            hs_hbm.at[b, idx, pl.ds(0, _HHID)], row_v.at[pl.ds(0, _HHID)], s2
        )
        g0.start()
        g1 = pltpu.make_async_copy(
            hs_hbm.at[b, idx, pl.ds(_HHID, _HHID)],
            row_v.at[pl.ds(_HHID, _HHID)],
            s3,
        )
        g1.start()
        g0.wait()
        o0 = pltpu.make_async_copy(
            row_v.at[pl.ds(0, _HHID)], out_hbm.at[b, pl.ds(0, _HHID)], s0
        )
        o0.start()
        g1.wait()
        o1 = pltpu.make_async_copy(
            row_v.at[pl.ds(_HHID, _HHID)],
            out_hbm.at[b, pl.ds(_HHID, _HHID)],
            s1,
        )
        o1.start()
        o0.wait()
        o1.wait()


def kernel(input_ids, hidden_states):
    mesh = plsc.VectorSubcoreMesh(
        core_axis_name="c", subcore_axis_name="s", num_cores=1,
    )
    k = pl.kernel(
        _sc_body,
        out_type=jax.ShapeDtypeStruct((_BATCH, _HID), jnp.float32),
        mesh=mesh,
        scratch_types=[
            pltpu.VMEM((_SEQ,), jnp.int32),
            pltpu.VMEM((_HID,), jnp.float32),
            pltpu.SemaphoreType.DMA,
            pltpu.SemaphoreType.DMA,
            pltpu.SemaphoreType.DMA,
            pltpu.SemaphoreType.DMA,
        ],
    )
    return k(input_ids.astype(jnp.int32), hidden_states)
